# baseline (speedup 1.0000x reference)
import functools

import jax
import jax.numpy as jnp
from jax import lax
from jax.experimental import pallas as pl
from jax.experimental.pallas import tpu as pltpu
from jax.experimental.pallas import tpu_sc as plsc

E = 8
T = 2048
D = 2048
F = 2048
BM = 256
NBLK = (2 * T + E * BM) // BM
NROWS = NBLK * BM
NC, NS, L = 2, 16, 16
NW = NC * NS
TPW = T // NW
CHT = 16
NCH = TPW // CHT
CUM = 512


def _route_body(x_ref, wg_ref, pos_ref, g_ref, be_ref, bv_ref,
                tr_ref, sl_ref, nx_ref, hn_ref, h_ref, m_ref):
    logits = jnp.dot(x_ref[...], wg_ref[...], preferred_element_type=jnp.float32)
    iota = lax.broadcasted_iota(jnp.int32, (T, E), 1)
    m1 = jnp.max(logits, axis=1, keepdims=True)
    i1 = jnp.min(jnp.where(logits == m1, iota, E), axis=1, keepdims=True)
    mask1 = iota == i1
    l2 = jnp.where(mask1, -1e30, logits)
    m2 = jnp.max(l2, axis=1, keepdims=True)
    i2 = jnp.min(jnp.where(l2 == m2, iota, E), axis=1, keepdims=True)
    mask2 = iota == i2
    z = jnp.exp(m2 - m1)
    w1 = 1.0 / (1.0 + z)
    w2 = w1 * z
    g_ref[...] = jnp.concatenate([w1, w2], axis=0)

    h_ref[...] = jnp.concatenate(
        [mask1.astype(jnp.float32), mask2.astype(jnp.float32)], axis=0)

    r = lax.broadcasted_iota(jnp.int32, (CUM, CUM), 0)
    c = lax.broadcasted_iota(jnp.int32, (CUM, CUM), 1)
    tri = (r > c).astype(jnp.float32)

    def step(i, tot):
        hc = h_ref[pl.ds(i * CUM, CUM), :]
        m_ref[pl.ds(i * CUM, CUM), :] = (
            jnp.dot(tri, hc, preferred_element_type=jnp.float32) + tot)
        return tot + jnp.sum(hc, axis=0, keepdims=True)

    tot = lax.fori_loop(0, 2 * T // CUM, step, jnp.zeros((1, E), jnp.float32))

    cnt = tot
    padded = jnp.ceil(cnt / BM) * BM
    er = lax.broadcasted_iota(jnp.int32, (E, E), 0)
    ec = lax.broadcasted_iota(jnp.int32, (E, E), 1)
    triu = (er < ec).astype(jnp.float32)
    off = jnp.dot(padded, triu, preferred_element_type=jnp.float32)

    h = h_ref[...]
    rank = jnp.sum(m_ref[...] * h, axis=1, keepdims=True)
    offsel = jnp.sum(h * off, axis=1, keepdims=True)
    pos_ref[...] = (rank + offsel).astype(jnp.int32)

    ends_blk = ((off + padded) / BM).astype(jnp.int32)
    bi = lax.broadcasted_iota(jnp.int32, (NBLK, E), 0)
    be = jnp.sum((bi >= ends_blk).astype(jnp.int32), axis=1, keepdims=True)
    be = jnp.minimum(be, E - 1)
    bl = lax.broadcasted_iota(jnp.int32, (NBLK, E), 1)
    oh = (bl == be).astype(jnp.float32)
    cnt_sel = jnp.sum(oh * cnt, axis=1, keepdims=True)
    off_sel = jnp.sum(oh * off, axis=1, keepdims=True)
    bid = lax.broadcasted_iota(jnp.int32, (NBLK, 1), 0)
    nrows = cnt_sel.astype(jnp.int32) - (bid * BM - off_sel.astype(jnp.int32))
    nrows = jnp.clip(nrows, 0, BM)
    bv = (nrows > 0).astype(jnp.int32)
    be_ref[...] = be
    bv_ref[...] = bv

    prev = jnp.concatenate([be[0:1] - 1, be[0 : NBLK - 1]], axis=0)
    trans = ((be != prev).astype(jnp.int32)) * bv
    tr_ref[...] = trans
    tl = (lax.broadcasted_iota(jnp.int32, (NBLK, NBLK), 0)
          >= lax.broadcasted_iota(jnp.int32, (NBLK, NBLK), 1)).astype(jnp.float32)
    incl = jnp.dot(tl, trans.astype(jnp.float32),
                   preferred_element_type=jnp.float32).astype(jnp.int32)
    sl_ref[...] = (incl - 1) & 1
    el = lax.broadcasted_iota(jnp.int32, (NBLK, E), 1)
    present = (cnt > 0.5)
    nxmask = (el > be) & present
    nxte = jnp.min(jnp.where(nxmask, el, E), axis=1, keepdims=True)
    hn_ref[...] = (nxte < E).astype(jnp.int32)
    nx_ref[...] = jnp.minimum(nxte, E - 1)


def _route_call(xf, w_gate):
    return pl.pallas_call(
        _route_body,
        grid=(1,),
        in_specs=[
            pl.BlockSpec((T, D), lambda i: (0, 0)),
            pl.BlockSpec((D, E), lambda i: (0, 0)),
        ],
        out_specs=[
            pl.BlockSpec((2 * T, 1), lambda i: (0, 0)),
            pl.BlockSpec((2 * T, 1), lambda i: (0, 0)),
        ] + [pl.BlockSpec((NBLK, 1), lambda i: (0, 0))] * 6,
        out_shape=[
            jax.ShapeDtypeStruct((2 * T, 1), jnp.int32),
            jax.ShapeDtypeStruct((2 * T, 1), jnp.float32),
        ] + [jax.ShapeDtypeStruct((NBLK, 1), jnp.int32)] * 6,
        scratch_shapes=[
            pltpu.VMEM((2 * T, E), jnp.float32),
            pltpu.VMEM((2 * T, E), jnp.float32),
        ],
    )(xf, w_gate)


def _dispatch_body(x_hbm, pa_hbm, pb_hbm, xs_hbm, ia_v, ib_v, buf, sa, sb):
    wid = lax.axis_index("s") * NC + lax.axis_index("c")
    pltpu.sync_copy(pa_hbm.at[pl.ds(wid * NCH, NCH)], ia_v)
    pltpu.sync_copy(pb_hbm.at[pl.ds(wid * NCH, NCH)], ib_v)

    def chunk(c, carry):
        tokbase = wid * TPW + c * CHT
        pltpu.sync_copy(x_hbm.at[pl.ds(tokbase, CHT)], buf)
        a = pltpu.async_copy(buf, xs_hbm.at[ia_v.at[c]], sa)
        b = pltpu.async_copy(buf, xs_hbm.at[ib_v.at[c]], sb)
        a.wait()
        b.wait()
        return carry

    lax.fori_loop(0, NCH, chunk, 0)


def _dispatch_sc(xf, posA, posB):
    k = pl.kernel(
        _dispatch_body,
        out_type=jax.ShapeDtypeStruct((NROWS, D), jnp.float32),
        mesh=plsc.VectorSubcoreMesh(
            core_axis_name="c", subcore_axis_name="s"),
        scratch_types=[
            pltpu.VMEM((NCH, CHT), jnp.int32),
            pltpu.VMEM((NCH, CHT), jnp.int32),
            pltpu.VMEM((CHT, D), jnp.float32),
            pltpu.SemaphoreType.DMA,
            pltpu.SemaphoreType.DMA,
        ],
    )
    return k(xf, posA, posB)


def _matmul_body(meta_ref, xs_ref, w_hbm, b_ref, ys_ref, wbuf, sems):
    b = pl.program_id(0)
    be = meta_ref[0, b]
    bv = meta_ref[1, b]
    slot = meta_ref[2, b]
    nxte = meta_ref[3, b]
    trans = meta_ref[4, b]
    hasnx = meta_ref[5, b]

    @pl.when(b == 0)
    def _():
        pltpu.make_async_copy(w_hbm.at[be], wbuf.at[slot], sems.at[slot]).start()

    @pl.when(trans > 0)
    def _():
        pltpu.make_async_copy(w_hbm.at[be], wbuf.at[slot], sems.at[slot]).wait()

        @pl.when(hasnx > 0)
        def _():
            pltpu.make_async_copy(
                w_hbm.at[nxte], wbuf.at[1 - slot], sems.at[1 - slot]).start()

    @pl.when(bv > 0)
    def _():
        ys_ref[...] = (
            jnp.dot(xs_ref[...], wbuf[slot], preferred_element_type=jnp.float32)
            + b_ref[0])


def _group_matmul(meta, xs, w_experts, b3):
    grid_spec = pltpu.PrefetchScalarGridSpec(
        num_scalar_prefetch=1,
        grid=(NBLK,),
        in_specs=[
            pl.BlockSpec((BM, D), lambda b, meta: (b, 0)),
            pl.BlockSpec(memory_space=pltpu.MemorySpace.HBM),
            pl.BlockSpec((1, 1, F), lambda b, meta: (meta[0, b], 0, 0)),
        ],
        out_specs=pl.BlockSpec((BM, F), lambda b, meta: (b, 0)),
        scratch_shapes=[
            pltpu.VMEM((2, D, F), jnp.float32),
            pltpu.SemaphoreType.DMA((2,)),
        ],
    )
    return pl.pallas_call(
        _matmul_body,
        grid_spec=grid_spec,
        out_shape=jax.ShapeDtypeStruct((NROWS, F), jnp.float32),
        compiler_params=pltpu.CompilerParams(
            dimension_semantics=("arbitrary",),
        ),
    )(meta, xs, w_experts, b3)


def _combine_body(ys_hbm, pa_hbm, pb_hbm, ga_hbm, gb_hbm, out_hbm,
                  ia_v, ib_v, ga_v, gb_v, bufa, bufb, obuf, sa, sb):
    wid = lax.axis_index("s") * NC + lax.axis_index("c")
    pltpu.sync_copy(pa_hbm.at[pl.ds(wid * NCH, NCH)], ia_v)
    pltpu.sync_copy(pb_hbm.at[pl.ds(wid * NCH, NCH)], ib_v)
    pltpu.sync_copy(ga_hbm.at[pl.ds(wid * NCH, NCH)], ga_v)
    pltpu.sync_copy(gb_hbm.at[pl.ds(wid * NCH, NCH)], gb_v)

    def chunk(c, carry):
        a = pltpu.async_copy(ys_hbm.at[ia_v.at[c]], bufa, sa)
        b = pltpu.async_copy(ys_hbm.at[ib_v.at[c]], bufb, sb)
        a.wait()
        b.wait()

        ga_row = ga_v[c, :]
        gb_row = gb_v[c, :]

        dn = lax.GatherDimensionNumbers(
            offset_dims=(), collapsed_slice_dims=(0,), start_index_map=(0,))

        def token(i, carry2):
            sel = jnp.full((L, 1), i, jnp.int32)
            ga = lax.gather(ga_row, sel, dn, slice_sizes=(1,),
                            mode=lax.GatherScatterMode.PROMISE_IN_BOUNDS)
            gb = lax.gather(gb_row, sel, dn, slice_sizes=(1,),
                            mode=lax.GatherScatterMode.PROMISE_IN_BOUNDS)

            def lane(j, carry3):
                for u in range(8):
                    o = j * 8 * L + u * L
                    obuf[i, pl.ds(o, L)] = (
                        ga * bufa[i, pl.ds(o, L)] + gb * bufb[i, pl.ds(o, L)])
                return carry3

            lax.fori_loop(0, F // (8 * L), lane, 0)
            return carry2

        lax.fori_loop(0, CHT, token, 0)
        pltpu.sync_copy(obuf, out_hbm.at[pl.ds(wid * TPW + c * CHT, CHT)])
        return carry

    lax.fori_loop(0, NCH, chunk, 0)


def _combine_sc(ys, posA, posB, gA, gB):
    k = pl.kernel(
        _combine_body,
        out_type=jax.ShapeDtypeStruct((T, F), jnp.float32),
        mesh=plsc.VectorSubcoreMesh(
            core_axis_name="c", subcore_axis_name="s"),
        scratch_types=[
            pltpu.VMEM((NCH, CHT), jnp.int32),
            pltpu.VMEM((NCH, CHT), jnp.int32),
            pltpu.VMEM((NCH, CHT), jnp.float32),
            pltpu.VMEM((NCH, CHT), jnp.float32),
            pltpu.VMEM((CHT, D), jnp.float32),
            pltpu.VMEM((CHT, D), jnp.float32),
            pltpu.VMEM((CHT, D), jnp.float32),
            pltpu.SemaphoreType.DMA,
            pltpu.SemaphoreType.DMA,
        ],
    )
    return k(ys, posA, posB, gA, gB)


def kernel(x, w_gate, w_experts, b_experts):
    B, S, _ = x.shape
    xf = x.reshape(T, D)
    pos2d, g2d, be2d, bv2d, tr2d, sl2d, nx2d, hn2d = _route_call(xf, w_gate)
    pos = pos2d.reshape(2, NW * NCH, CHT)
    posA, posB = pos[0], pos[1]
    g = g2d.reshape(2, NW * NCH, CHT)
    gA, gB = g[0], g[1]
    meta = jnp.concatenate(
        [be2d, bv2d, sl2d, nx2d, tr2d, hn2d,
         jnp.zeros((NBLK, 2), jnp.int32)], axis=1).T
    xs = _dispatch_sc(xf, posA, posB)
    ys = _group_matmul(meta, xs, w_experts, b_experts.reshape(E, 1, F))
    out = _combine_sc(ys, posA, posB, gA, gB)
    return out.reshape(B, S, F)


# speedup vs baseline: 1.2473x; 1.2473x over previous
import functools

import jax
import jax.numpy as jnp
from jax import lax
from jax.experimental import pallas as pl
from jax.experimental.pallas import tpu as pltpu
from jax.experimental.pallas import tpu_sc as plsc

E = 8
T = 2048
D = 2048
F = 2048
BM = 256
NBLK = (2 * T + E * BM) // BM
NROWS = NBLK * BM
NC, NS, L = 2, 16, 16
NW = NC * NS
TPW = T // NW
CHT = 16
NCH = TPW // CHT
CUM = 512


def _route_body(x_ref, wg_ref, pos_ref, g_ref, be_ref, bv_ref,
                tr_ref, sl_ref, nx_ref, hn_ref, h_ref, m_ref):
    logits = jnp.dot(x_ref[...], wg_ref[...], preferred_element_type=jnp.float32)
    iota = lax.broadcasted_iota(jnp.int32, (T, E), 1)
    m1 = jnp.max(logits, axis=1, keepdims=True)
    i1 = jnp.min(jnp.where(logits == m1, iota, E), axis=1, keepdims=True)
    mask1 = iota == i1
    l2 = jnp.where(mask1, -1e30, logits)
    m2 = jnp.max(l2, axis=1, keepdims=True)
    i2 = jnp.min(jnp.where(l2 == m2, iota, E), axis=1, keepdims=True)
    mask2 = iota == i2
    z = jnp.exp(m2 - m1)
    w1 = 1.0 / (1.0 + z)
    w2 = w1 * z
    g_ref[...] = jnp.concatenate([w1, w2], axis=0)

    h_ref[...] = jnp.concatenate(
        [mask1.astype(jnp.float32), mask2.astype(jnp.float32)], axis=0)

    r = lax.broadcasted_iota(jnp.int32, (CUM, CUM), 0)
    c = lax.broadcasted_iota(jnp.int32, (CUM, CUM), 1)
    tri = (r > c).astype(jnp.float32)

    def step(i, tot):
        hc = h_ref[pl.ds(i * CUM, CUM), :]
        m_ref[pl.ds(i * CUM, CUM), :] = (
            jnp.dot(tri, hc, preferred_element_type=jnp.float32) + tot)
        return tot + jnp.sum(hc, axis=0, keepdims=True)

    tot = lax.fori_loop(0, 2 * T // CUM, step, jnp.zeros((1, E), jnp.float32))

    cnt = tot
    padded = jnp.ceil(cnt / BM) * BM
    er = lax.broadcasted_iota(jnp.int32, (E, E), 0)
    ec = lax.broadcasted_iota(jnp.int32, (E, E), 1)
    triu = (er < ec).astype(jnp.float32)
    off = jnp.dot(padded, triu, preferred_element_type=jnp.float32)

    h = h_ref[...]
    rank = jnp.sum(m_ref[...] * h, axis=1, keepdims=True)
    offsel = jnp.sum(h * off, axis=1, keepdims=True)
    pos_ref[...] = (rank + offsel).astype(jnp.int32)

    ends_blk = ((off + padded) / BM).astype(jnp.int32)
    bi = lax.broadcasted_iota(jnp.int32, (NBLK, E), 0)
    be = jnp.sum((bi >= ends_blk).astype(jnp.int32), axis=1, keepdims=True)
    be = jnp.minimum(be, E - 1)
    bl = lax.broadcasted_iota(jnp.int32, (NBLK, E), 1)
    oh = (bl == be).astype(jnp.float32)
    cnt_sel = jnp.sum(oh * cnt, axis=1, keepdims=True)
    off_sel = jnp.sum(oh * off, axis=1, keepdims=True)
    bid = lax.broadcasted_iota(jnp.int32, (NBLK, 1), 0)
    nrows = cnt_sel.astype(jnp.int32) - (bid * BM - off_sel.astype(jnp.int32))
    nrows = jnp.clip(nrows, 0, BM)
    bv = (nrows > 0).astype(jnp.int32)
    be_ref[...] = be
    bv_ref[...] = bv

    prev = jnp.concatenate([be[0:1] - 1, be[0 : NBLK - 1]], axis=0)
    trans = ((be != prev).astype(jnp.int32)) * bv
    tr_ref[...] = trans
    tl = (lax.broadcasted_iota(jnp.int32, (NBLK, NBLK), 0)
          >= lax.broadcasted_iota(jnp.int32, (NBLK, NBLK), 1)).astype(jnp.float32)
    incl = jnp.dot(tl, trans.astype(jnp.float32),
                   preferred_element_type=jnp.float32).astype(jnp.int32)
    sl_ref[...] = (incl - 1) & 1
    el = lax.broadcasted_iota(jnp.int32, (NBLK, E), 1)
    present = (cnt > 0.5)
    nxmask = (el > be) & present
    nxte = jnp.min(jnp.where(nxmask, el, E), axis=1, keepdims=True)
    hn_ref[...] = (nxte < E).astype(jnp.int32)
    nx_ref[...] = jnp.minimum(nxte, E - 1)


def _route_call(xf, w_gate):
    return pl.pallas_call(
        _route_body,
        grid=(1,),
        in_specs=[
            pl.BlockSpec((T, D), lambda i: (0, 0)),
            pl.BlockSpec((D, E), lambda i: (0, 0)),
        ],
        out_specs=[
            pl.BlockSpec((2 * T, 1), lambda i: (0, 0)),
            pl.BlockSpec((2 * T, 1), lambda i: (0, 0)),
        ] + [pl.BlockSpec((NBLK, 1), lambda i: (0, 0))] * 6,
        out_shape=[
            jax.ShapeDtypeStruct((2 * T, 1), jnp.int32),
            jax.ShapeDtypeStruct((2 * T, 1), jnp.float32),
        ] + [jax.ShapeDtypeStruct((NBLK, 1), jnp.int32)] * 6,
        scratch_shapes=[
            pltpu.VMEM((2 * T, E), jnp.float32),
            pltpu.VMEM((2 * T, E), jnp.float32),
        ],
    )(xf, w_gate)


def _dispatch_body(x_hbm, pa_hbm, pb_hbm, xs_hbm, ia_v, ib_v, buf, sa, sb):
    wid = lax.axis_index("s") * NC + lax.axis_index("c")
    pltpu.sync_copy(pa_hbm.at[pl.ds(wid * NCH, NCH)], ia_v)
    pltpu.sync_copy(pb_hbm.at[pl.ds(wid * NCH, NCH)], ib_v)

    def chunk(c, carry):
        tokbase = wid * TPW + c * CHT
        pltpu.sync_copy(x_hbm.at[pl.ds(tokbase, CHT)], buf)
        a = pltpu.async_copy(buf, xs_hbm.at[ia_v.at[c]], sa)
        b = pltpu.async_copy(buf, xs_hbm.at[ib_v.at[c]], sb)
        a.wait()
        b.wait()
        return carry

    lax.fori_loop(0, NCH, chunk, 0)


def _dispatch_sc(xf, posA, posB):
    k = pl.kernel(
        _dispatch_body,
        out_type=jax.ShapeDtypeStruct((NROWS, D), jnp.float32),
        mesh=plsc.VectorSubcoreMesh(
            core_axis_name="c", subcore_axis_name="s"),
        scratch_types=[
            pltpu.VMEM((NCH, CHT), jnp.int32),
            pltpu.VMEM((NCH, CHT), jnp.int32),
            pltpu.VMEM((CHT, D), jnp.float32),
            pltpu.SemaphoreType.DMA,
            pltpu.SemaphoreType.DMA,
        ],
    )
    return k(xf, posA, posB)


def _matmul_body(meta_ref, xs_ref, w_hbm, b_ref, ys_ref, wbuf, sems):
    b = pl.program_id(0)
    be = meta_ref[0, b]
    bv = meta_ref[1, b]
    slot = meta_ref[2, b]
    nxte = meta_ref[3, b]
    trans = meta_ref[4, b]
    hasnx = meta_ref[5, b]

    @pl.when(b == 0)
    def _():
        pltpu.make_async_copy(w_hbm.at[be], wbuf.at[slot], sems.at[slot]).start()

    @pl.when(trans > 0)
    def _():
        pltpu.make_async_copy(w_hbm.at[be], wbuf.at[slot], sems.at[slot]).wait()

        @pl.when(hasnx > 0)
        def _():
            pltpu.make_async_copy(
                w_hbm.at[nxte], wbuf.at[1 - slot], sems.at[1 - slot]).start()

    @pl.when(bv > 0)
    def _():
        ys_ref[...] = (
            jnp.dot(xs_ref[...], wbuf[slot], preferred_element_type=jnp.float32)
            + b_ref[0])


def _group_matmul(meta, xs, w_experts, b3):
    grid_spec = pltpu.PrefetchScalarGridSpec(
        num_scalar_prefetch=1,
        grid=(NBLK,),
        in_specs=[
            pl.BlockSpec((BM, D), lambda b, meta: (b, 0)),
            pl.BlockSpec(memory_space=pltpu.MemorySpace.HBM),
            pl.BlockSpec((1, 1, F), lambda b, meta: (meta[0, b], 0, 0)),
        ],
        out_specs=pl.BlockSpec((BM, F), lambda b, meta: (b, 0)),
        scratch_shapes=[
            pltpu.VMEM((2, D, F), jnp.float32),
            pltpu.SemaphoreType.DMA((2,)),
        ],
    )
    return pl.pallas_call(
        _matmul_body,
        grid_spec=grid_spec,
        out_shape=jax.ShapeDtypeStruct((NROWS, F), jnp.float32),
        compiler_params=pltpu.CompilerParams(
            dimension_semantics=("arbitrary",),
        ),
    )(meta, xs, w_experts, b3)


def _combine_body(ys_hbm, pa_hbm, pb_hbm, ga_hbm, gb_hbm, out_hbm,
                  ia_v, ib_v, ga_v, gb_v, bufa, bufb, obuf, sa, sb):
    wid = lax.axis_index("s") * NC + lax.axis_index("c")
    pltpu.sync_copy(pa_hbm.at[pl.ds(wid * NCH, NCH)], ia_v)
    pltpu.sync_copy(pb_hbm.at[pl.ds(wid * NCH, NCH)], ib_v)
    pltpu.sync_copy(ga_hbm.at[pl.ds(wid * NCH, NCH)], ga_v)
    pltpu.sync_copy(gb_hbm.at[pl.ds(wid * NCH, NCH)], gb_v)

    def chunk(c, carry):
        a = pltpu.async_copy(ys_hbm.at[ia_v.at[c]], bufa, sa)
        b = pltpu.async_copy(ys_hbm.at[ib_v.at[c]], bufb, sb)
        a.wait()
        b.wait()

        ga_row = ga_v[c, :]
        gb_row = gb_v[c, :]

        dn = lax.GatherDimensionNumbers(
            offset_dims=(), collapsed_slice_dims=(0,), start_index_map=(0,))

        def token(i, carry2):
            sel = jnp.full((L, 1), i, jnp.int32)
            ga = lax.gather(ga_row, sel, dn, slice_sizes=(1,),
                            mode=lax.GatherScatterMode.PROMISE_IN_BOUNDS)
            gb = lax.gather(gb_row, sel, dn, slice_sizes=(1,),
                            mode=lax.GatherScatterMode.PROMISE_IN_BOUNDS)

            @plsc.parallel_loop(0, F, step=L, unroll=4)
            def _lane(o):
                obuf[i, pl.ds(o, L)] = (
                    ga * bufa[i, pl.ds(o, L)] + gb * bufb[i, pl.ds(o, L)])

            return carry2

        lax.fori_loop(0, CHT, token, 0)
        pltpu.sync_copy(obuf, out_hbm.at[pl.ds(wid * TPW + c * CHT, CHT)])
        return carry

    lax.fori_loop(0, NCH, chunk, 0)


def _combine_sc(ys, posA, posB, gA, gB):
    k = pl.kernel(
        _combine_body,
        out_type=jax.ShapeDtypeStruct((T, F), jnp.float32),
        mesh=plsc.VectorSubcoreMesh(
            core_axis_name="c", subcore_axis_name="s"),
        scratch_types=[
            pltpu.VMEM((NCH, CHT), jnp.int32),
            pltpu.VMEM((NCH, CHT), jnp.int32),
            pltpu.VMEM((NCH, CHT), jnp.float32),
            pltpu.VMEM((NCH, CHT), jnp.float32),
            pltpu.VMEM((CHT, D), jnp.float32),
            pltpu.VMEM((CHT, D), jnp.float32),
            pltpu.VMEM((CHT, D), jnp.float32),
            pltpu.SemaphoreType.DMA,
            pltpu.SemaphoreType.DMA,
        ],
    )
    return k(ys, posA, posB, gA, gB)


def kernel(x, w_gate, w_experts, b_experts):
    B, S, _ = x.shape
    xf = x.reshape(T, D)
    pos2d, g2d, be2d, bv2d, tr2d, sl2d, nx2d, hn2d = _route_call(xf, w_gate)
    pos = pos2d.reshape(2, NW * NCH, CHT)
    posA, posB = pos[0], pos[1]
    g = g2d.reshape(2, NW * NCH, CHT)
    gA, gB = g[0], g[1]
    meta = jnp.concatenate(
        [be2d, bv2d, sl2d, nx2d, tr2d, hn2d,
         jnp.zeros((NBLK, 2), jnp.int32)], axis=1).T
    xs = _dispatch_sc(xf, posA, posB)
    ys = _group_matmul(meta, xs, w_experts, b_experts.reshape(E, 1, F))
    out = _combine_sc(ys, posA, posB, gA, gB)
    return out.reshape(B, S, F)
